# Initial kernel scaffold; baseline (speedup 1.0000x reference)
import jax
import jax.numpy as jnp
from jax.experimental import pallas as pl


def kernel(x, w1, b1, w2, b2, w3, b3, w4, b4, w5, b5):
    raise NotImplementedError("write your pallas kernel here")


# speedup vs baseline: 28.8091x; 28.8091x over previous
import numpy as np

import jax
import jax.numpy as jnp
from jax.experimental import pallas as pl
from jax.experimental.pallas import tpu as pltpu

_F32 = jnp.float32


def _cdiv(a, b):
    return -(-a // b)


def _band_maps(kh, kw, c_in, c_out, w_in, pw):
    rl = w_in * c_in
    rlp = 128 * _cdiv(rl, 128)
    rows = kh * rlp
    r = np.arange(rows)[:, None]
    col = np.arange(256)[None, :]
    i = r // rlp
    lr = r % rlp
    x = lr // c_in
    ic = lr % c_in
    p = col // 128
    l = col % 128
    pc = l // c_out
    oc = l % c_out
    j = x - (2 * pc + p)
    mask = (lr < rl) & (l < pw * c_out) & (j >= 0) & (j < kw)
    flat = np.where(mask, ((oc * c_in + ic) * kh + i) * kw + np.clip(j, 0, kw - 1), 0)
    return flat.astype(np.int32), mask


_W1_IDX, _W1_MASK = _band_maps(kh=5, kw=5, c_in=1, c_out=6, w_in=32, pw=14)
_W2_IDX, _W2_MASK = _band_maps(kh=5, kw=5, c_in=6, c_out=16, w_in=14, pw=5)

_r = np.arange(5 * 32)[:, None]
_c = np.arange(256)[None, :]
_i1 = _r // 32
_x1 = _r % 32
_p1 = _c // 128
_l1 = _c % 128
_pc1 = _l1 // 6
_oc1 = _l1 % 6
_j1 = _x1 - (2 * _pc1 + _p1)
_W1_MASK = (_l1 < 84) & (_j1 >= 0) & (_j1 < 5)
_W1_IDX = np.where(_W1_MASK, (_oc1 * 5 + _i1) * 5 + np.clip(_j1, 0, 4), 0).astype(np.int32)

_LANE = np.arange(128)
_B1_MASK = _LANE < 84
_B1_IDX = np.where(_B1_MASK, _LANE % 6, 0).astype(np.int32)
_B2_MASK = _LANE < 80
_B2_IDX = np.where(_B2_MASK, _LANE % 16, 0).astype(np.int32)


def _fused_kernel(x_ref, w1_ref, c1b_ref, w2_ref, c2b_ref, w3_ref, b3_ref,
                  w4_ref, b4_ref, w5_ref, b5_ref, o_ref):
    tb = x_ref.shape[0]
    x = x_ref[...]

    acc = None
    for i in range(5):
        lhs = x[:, i:i + 28, :].reshape(tb * 28, 32)
        part = jnp.dot(lhs, w1_ref[i * 32:(i + 1) * 32, :],
                       preferred_element_type=_F32)
        acc = part if acc is None else acc + part
    acc = acc.reshape(tb, 28, 256)
    acc = jnp.maximum(acc[:, :, 0:128], acc[:, :, 128:256])
    acc = acc.reshape(tb, 14, 2, 128).max(axis=2)
    a1 = jnp.maximum(acc + c1b_ref[...], 0.0)

    acc = None
    for i in range(5):
        lhs = a1[:, i:i + 10, :].reshape(tb * 10, 128)
        part = jnp.dot(lhs, w2_ref[i * 128:(i + 1) * 128, :],
                       preferred_element_type=_F32)
        acc = part if acc is None else acc + part
    acc = acc.reshape(tb, 10, 256)
    acc = jnp.maximum(acc[:, :, 0:128], acc[:, :, 128:256])
    acc = acc.reshape(tb, 5, 2, 128).max(axis=2)
    a2 = jnp.maximum(acc + c2b_ref[...], 0.0)

    h = None
    for i in range(5):
        part = jnp.dot(a2[:, i, :], w3_ref[i * 128:(i + 1) * 128, :],
                       preferred_element_type=_F32)
        h = part if h is None else h + part
    h = jnp.maximum(h + b3_ref[...], 0.0)
    h = jnp.dot(h, w4_ref[...], preferred_element_type=_F32)
    h = jnp.maximum(h + b4_ref[...], 0.0)
    h = jnp.dot(h, w5_ref[...], preferred_element_type=_F32)
    o_ref[...] = (h + b5_ref[...]).astype(o_ref.dtype)


def kernel(x, w1, b1, w2, b2, w3, b3, w4, b4, w5, b5):
    B = x.shape[0]
    xs = x.reshape(B, 32, 32).astype(_F32)

    w1m = jnp.where(jnp.asarray(_W1_MASK),
                    w1.reshape(-1).astype(_F32)[jnp.asarray(_W1_IDX)], 0.0)
    w2m = jnp.where(jnp.asarray(_W2_MASK),
                    w2.reshape(-1)
                      .astype(_F32)[jnp.asarray(_W2_IDX)], 0.0)
    c1b = jnp.where(jnp.asarray(_B1_MASK),
                    b1.astype(_F32)[jnp.asarray(_B1_IDX)], 0.0).reshape(1, 128)
    c2b = jnp.where(jnp.asarray(_B2_MASK),
                    b2.astype(_F32)[jnp.asarray(_B2_IDX)], 0.0).reshape(1, 128)

    w3t = jnp.transpose(w3, (2, 3, 1, 0)).reshape(5, 80, 120).astype(_F32)
    w3m = jnp.pad(w3t, ((0, 0), (0, 48), (0, 0))).reshape(640, 120)
    w4t = w4.T.astype(_F32)
    w5t = w5.T.astype(_F32)
    b3r = b3.reshape(1, 120).astype(_F32)
    b4r = b4.reshape(1, 84).astype(_F32)
    b5r = b5.reshape(1, 10).astype(_F32)

    tb = 256
    nb = _cdiv(B, tb)
    b_pad = nb * tb
    if b_pad != B:
        xs = jnp.pad(xs, ((0, b_pad - B), (0, 0), (0, 0)))

    out = pl.pallas_call(
        _fused_kernel,
        out_shape=jax.ShapeDtypeStruct((b_pad, 10), _F32),
        grid_spec=pltpu.PrefetchScalarGridSpec(
            num_scalar_prefetch=0,
            grid=(nb,),
            in_specs=[
                pl.BlockSpec((tb, 32, 32), lambda m: (m, 0, 0)),
                pl.BlockSpec((160, 256), lambda m: (0, 0)),
                pl.BlockSpec((1, 128), lambda m: (0, 0)),
                pl.BlockSpec((640, 256), lambda m: (0, 0)),
                pl.BlockSpec((1, 128), lambda m: (0, 0)),
                pl.BlockSpec((640, 120), lambda m: (0, 0)),
                pl.BlockSpec((1, 120), lambda m: (0, 0)),
                pl.BlockSpec((120, 84), lambda m: (0, 0)),
                pl.BlockSpec((1, 84), lambda m: (0, 0)),
                pl.BlockSpec((84, 10), lambda m: (0, 0)),
                pl.BlockSpec((1, 10), lambda m: (0, 0)),
            ],
            out_specs=pl.BlockSpec((tb, 10), lambda m: (m, 0)),
        ),
        compiler_params=pltpu.CompilerParams(
            dimension_semantics=("parallel",),
            vmem_limit_bytes=64 * 1024 * 1024,
        ),
        cost_estimate=pl.CostEstimate(
            flops=2 * b_pad * (28 * 160 * 256 + 10 * 640 * 256 + 640 * 120
                               + 120 * 84 + 84 * 10),
            transcendentals=0,
            bytes_accessed=4 * (b_pad * 32 * 32 + b_pad * 10),
        ),
    )(xs, w1m, c1b, w2m, c2b, w3m, b3r, w4t, b4r, w5t, b5r)
    return out[:B]
